# Initial kernel scaffold; baseline (speedup 1.0000x reference)
import jax
import jax.numpy as jnp
from jax.experimental import pallas as pl


def kernel(x, params):
    raise NotImplementedError("write your pallas kernel here")


# speedup vs baseline: 1.4219x; 1.4219x over previous
import functools
import math

import jax
import jax.numpy as jnp
from jax.experimental import pallas as pl

import numpy as np

_SQRT_HALF = np.float32(math.sqrt(0.5))

_ERFC_P = [2.326819970068386e-2, -1.387039388740657e-1, 3.687424674597105e-1,
           -5.824733027278666e-1, 6.210004621745983e-1, -4.944515323274145e-1,
           3.404879937665872e-1, -2.741127028184656e-1, 5.638259427386472e-1]
_ERFC_R = [-1.047766399936249e+1, 1.297719955372516e+1, -7.495518717768503e+0,
           2.921019019210786e+0, -1.015265279202700e+0, 4.218463358204948e-1,
           -2.820767439740514e-1, 5.641895067754075e-1]
_ERF_T = [7.853861353153693e-5, -8.010193625184903e-4, 5.188327685732524e-3,
          -2.685381193529856e-2, 1.128358514861418e-1, -3.761262582423300e-1,
          1.128379165726710e+0]


def _poly(y, coef):
    p = jnp.full_like(y, np.float32(coef[0]))
    for c in coef[1:]:
        p = p * y + np.float32(c)
    return p


def _erfc(x):
    ax = jnp.abs(x)
    z = jnp.exp(-x * x)
    q = 1.0 / ax
    y = q * q
    p = jnp.where(ax < 2.0, _poly(y, _ERFC_P), _poly(y, _ERFC_R))
    ya = (z * q) * p
    res = jnp.where(x < 0.0, 2.0 - ya, ya)
    erf_small = x * _poly(x * x, _ERF_T)
    return jnp.where(ax < 1.0, 1.0 - erf_small, res)


def _gelu(x):
    return (0.5 * x) * _erfc(-x * _SQRT_HALF)


def _ln(x, g, b):
    m = jnp.mean(x, axis=-1, keepdims=True)
    v = jnp.var(x, axis=-1, keepdims=True)
    return (x - m) / jnp.sqrt(v + 1e-5) * g + b


def _lb(src, dww, dwb, pwt, pwb):
    n2, w, c = src.shape
    n = n2 - 2
    zc = jnp.zeros((n2, 1, c), jnp.float32)
    p = jnp.concatenate([zc, src, zc], axis=1)
    p = p.astype(jnp.bfloat16).astype(jnp.float32)
    acc = jnp.zeros((n, w, c), jnp.float32)
    for a in range(3):
        for b in range(3):
            wt = dww[a * 3 + b].astype(jnp.bfloat16).astype(jnp.float32)
            acc = acc + p[a:a + n, b:b + w, :] * wt[None, None, :]
    act = _gelu(acc + dwb[0][None, None, :])
    y = jnp.dot(act.reshape(n * w, c), pwt[...],
                preferred_element_type=jnp.float32) + pwb[...]
    return y.reshape(n, w, c)


def _row(ref, k, total, phase=None):
    kc = jnp.clip(k, 0, total - 1)
    r = ref[pl.ds(kc, 1)] if phase is None else ref[pl.ds(kc, 1), phase]
    valid = jnp.logical_and(k >= 0, k < total)
    return jnp.where(valid, r, 0.0)


def _window(ref, start, n, total, phase=None):
    mid = ref[pl.ds(start, n)] if phase is None else ref[pl.ds(start, n), phase]
    return jnp.concatenate([
        _row(ref, start - 2, total, phase),
        _row(ref, start - 1, total, phase),
        mid,
        _row(ref, start + n, total, phase),
        _row(ref, start + n + 1, total, phase),
    ], axis=0)


def _window1(ref, start, n, total, phase=None):
    mid = ref[pl.ds(start, n)] if phase is None else ref[pl.ds(start, n), phase]
    return jnp.concatenate([
        _row(ref, start - 1, total, phase),
        mid,
        _row(ref, start + n, total, phase),
    ], axis=0)


def _zero_edges(x, i, ng):
    top = jnp.where(i == 0, 0.0, x[:1])
    bot = jnp.where(i == ng - 1, 0.0, x[-1:])
    return jnp.concatenate([top, x[1:-1], bot], axis=0)


def _lift_pair(u_ext, v_ext, pp, pu, i, ng):
    h_ext = v_ext - _lb(u_ext, *pp)
    h_ext = _zero_edges(h_ext, i, ng)
    l = u_ext[2:-2] + _lb(h_ext, *pu)
    return h_ext, l


def _inv_pair(lo_ext, hi_ext, pp, pu, i, ng):
    ev_ext = lo_ext - _lb(hi_ext, *pu)
    ev_ext = _zero_edges(ev_ext, i, ng)
    od = hi_ext[2:-2] + _lb(ev_ext, *pp)
    return ev_ext, od


def _wts(p, c):
    dww = jnp.transpose(p['dw_w'][:, 0], (1, 2, 0)).reshape(9, c)
    pwt = jnp.transpose(p['pw_w'][:, :, 0, 0])
    return dww, p['dw_b'].reshape(1, c), pwt, p['pw_b'].reshape(1, c)


def _full(shape):
    return pl.BlockSpec(shape, lambda i: tuple(0 for _ in shape))


def _hdwt_body(x_ref, w_ref, b_ref,
               pdww_ref, pdwb_ref, ppwt_ref, ppwb_ref,
               udww_ref, udwb_ref, upwt_ref, upwb_ref,
               h_ref, l_ref, *, tile, nrows, din, c):
    i = pl.program_id(0)
    ng = pl.num_programs(0)
    start = i * tile
    w = h_ref.shape[1]

    def proj(win, nr, lo):
        t = jnp.dot(win[..., lo:lo + din].reshape(nr * w, din), w_ref[...],
                    preferred_element_type=jnp.float32) + b_ref[...]
        return t.reshape(nr, w, c)

    xwin4 = _window(x_ref, start, tile, nrows)
    te = proj(xwin4, tile + 4, 0)
    to = proj(xwin4[1:-1], tile + 2, din)
    top = jnp.where(i == 0, 0.0, te[:2])
    bot = jnp.where(i == ng - 1, 0.0, te[-2:])
    te = jnp.concatenate([top, te[2:-2], bot], axis=0)
    to = _zero_edges(to, i, ng)
    pp = (pdww_ref, pdwb_ref, ppwt_ref, ppwb_ref)
    pu = (udww_ref, udwb_ref, upwt_ref, upwb_ref)
    h_ext, l = _lift_pair(te, to, pp, pu, i, ng)
    h_ref[...] = h_ext[1:-1]
    l_ref[...] = l


def _hdwt(x2, p, tile=28):
    nrows, w, din2 = x2.shape
    din = din2 // 2
    c = p['x_trans_w'].shape[1]
    grid = nrows // tile
    pw = _wts(p['dwt']['P_h'], c)
    uw = _wts(p['dwt']['U_h'], c)
    body = functools.partial(_hdwt_body, tile=tile, nrows=nrows, din=din, c=c)
    return pl.pallas_call(
        body,
        grid=(grid,),
        in_specs=[
            _full((nrows, w, din2)),
            _full((din, c)), _full((1, c)),
            _full((9, c)), _full((1, c)), _full((c, c)), _full((1, c)),
            _full((9, c)), _full((1, c)), _full((c, c)), _full((1, c)),
        ],
        out_specs=[
            pl.BlockSpec((tile, w, c), lambda i: (i, 0, 0)),
            pl.BlockSpec((tile, w, c), lambda i: (i, 0, 0)),
        ],
        out_shape=(
            jax.ShapeDtypeStruct((nrows, w, c), jnp.float32),
            jax.ShapeDtypeStruct((nrows, w, c), jnp.float32),
        ),
    )(x2, p['x_trans_w'], p['x_trans_b'].reshape(1, c), *pw, *uw)


def _vdwt_body(lv_ref, hv_ref,
               pdww_ref, pdwb_ref, ppwt_ref, ppwb_ref,
               udww_ref, udwb_ref, upwt_ref, upwb_ref,
               ll_ref, th_ref, *, tile, nrows):
    i = pl.program_id(0)
    ng = pl.num_programs(0)
    start = i * tile
    pp = (pdww_ref, pdwb_ref, ppwt_ref, ppwb_ref)
    pu = (udww_ref, udwb_ref, upwt_ref, upwb_ref)

    ev_l = _window(lv_ref, start, tile, nrows, 0)
    od_l = _window1(lv_ref, start, tile, nrows, 1)
    h_ll_ext, ll = _lift_pair(ev_l, od_l, pp, pu, i, ng)

    ev_h = _window(hv_ref, start, tile, nrows, 0)
    od_h = _window1(hv_ref, start, tile, nrows, 1)
    h_hh_ext, lh = _lift_pair(ev_h, od_h, pp, pu, i, ng)

    ll_ref[...] = ll
    th_ref[...] = jnp.concatenate([h_ll_ext[1:-1], lh, h_hh_ext[1:-1]], axis=-1)


def _vdwt(lv, hv, p, tile=28):
    nrows, _, w, c = lv.shape
    grid = nrows // tile
    pw = _wts(p['dwt']['P_v'], c)
    uw = _wts(p['dwt']['U_v'], c)
    body = functools.partial(_vdwt_body, tile=tile, nrows=nrows)
    return pl.pallas_call(
        body,
        grid=(grid,),
        in_specs=[
            _full((nrows, 2, w, c)), _full((nrows, 2, w, c)),
            _full((9, c)), _full((1, c)), _full((c, c)), _full((1, c)),
            _full((9, c)), _full((1, c)), _full((c, c)), _full((1, c)),
        ],
        out_specs=[
            pl.BlockSpec((tile, w, c), lambda i: (i, 0, 0)),
            pl.BlockSpec((tile, w, 3 * c), lambda i: (i, 0, 0)),
        ],
        out_shape=(
            jax.ShapeDtypeStruct((nrows, w, c), jnp.float32),
            jax.ShapeDtypeStruct((nrows, w, 3 * c), jnp.float32),
        ),
    )(lv, hv, *pw, *uw)


def _prep_body(dl_ref, lg_ref, lb_ref, kw_ref, kb_ref,
               dh_ref, hg_ref, hb_ref, khw_ref, khb_ref,
               dlo_ref, klo_ref, dho_ref, kho_ref):
    d = _ln(dl_ref[...], lg_ref[...], lb_ref[...])
    dlo_ref[...] = d
    klo_ref[...] = jnp.dot(d, kw_ref[...], preferred_element_type=jnp.float32) + kb_ref[...]
    dh = _ln(dh_ref[...], hg_ref[...], hb_ref[...])
    dho_ref[...] = dh
    kho_ref[...] = jnp.dot(dh, khw_ref[...], preferred_element_type=jnp.float32) + khb_ref[...]


def _dict_prep(p):
    m, c = p['dict_low'].shape
    em = p['dict_high'].shape[0] * p['dict_high'].shape[1]
    c3 = p['dict_high'].shape[2]
    out_shape = (
        jax.ShapeDtypeStruct((m, c), jnp.float32),
        jax.ShapeDtypeStruct((m, c), jnp.float32),
        jax.ShapeDtypeStruct((em, c3), jnp.float32),
        jax.ShapeDtypeStruct((em, c3), jnp.float32),
    )
    return pl.pallas_call(_prep_body, out_shape=out_shape)(
        p['dict_low'], p['ln_dict_low_g'].reshape(1, c), p['ln_dict_low_b'].reshape(1, c),
        p['k_low_w'], p['k_low_b'].reshape(1, c),
        p['dict_high'].reshape(em, c3),
        p['ln_dict_high_g'].reshape(1, c3), p['ln_dict_high_b'].reshape(1, c3),
        p['k_high_w'], p['k_high_b'].reshape(1, c3),
    )


def _token_body(tl_ref, th_ref,
                lng_ref, lnb_ref, qw_ref, qb_ref, kl_ref, dl_ref,
                w1_ref, b1_ref, w2_ref, b2_ref, w3_ref, b3_ref,
                hg_ref, hb_ref, qhw_ref, qhb_ref, kh_ref, dh_ref,
                tlo_ref, tho_ref, *, n_exp, m_dict):
    tl = tl_ref[...]
    th = th_ref[...]
    tn, c = tl.shape
    c3 = th.shape[1]
    q = jnp.dot(_ln(tl, lng_ref[...], lnb_ref[...]), qw_ref[...],
                preferred_element_type=jnp.float32) + qb_ref[...]
    s = jnp.dot(q, kl_ref[...].T, preferred_element_type=jnp.float32) * (c ** -0.5)
    s = s - jnp.max(s, axis=-1, keepdims=True)
    e = jnp.exp(s)
    a = e / jnp.sum(e, axis=-1, keepdims=True)
    tl_new = tl + jnp.dot(a, dl_ref[...], preferred_element_type=jnp.float32)
    tlo_ref[...] = tl_new
    comb = jnp.concatenate([th, tl_new], axis=-1)
    hdn = _gelu(jnp.dot(comb, w1_ref[...], preferred_element_type=jnp.float32) + b1_ref[...])
    hdn = _gelu(jnp.dot(hdn, w2_ref[...], preferred_element_type=jnp.float32) + b2_ref[...])
    logits = jnp.dot(hdn, w3_ref[...], preferred_element_type=jnp.float32) + b3_ref[...]
    logits = logits - jnp.max(logits, axis=-1, keepdims=True)
    eg = jnp.exp(logits)
    gates = eg / jnp.sum(eg, axis=-1, keepdims=True)
    idx = jax.lax.broadcasted_iota(jnp.int32, (tn, n_exp), 1)
    m1 = jnp.max(gates, axis=-1, keepdims=True)
    i1 = -jnp.max(jnp.where(gates == m1, -idx, -(2 ** 30)), axis=-1, keepdims=True)
    g_wo1 = jnp.where(idx == i1, -jnp.float32(jnp.inf), gates)
    m2 = jnp.max(g_wo1, axis=-1, keepdims=True)
    i2 = -jnp.max(jnp.where(g_wo1 == m2, -idx, -(2 ** 30)), axis=-1, keepdims=True)
    mask = jnp.logical_or(idx == i1, idx == i2)
    g = jnp.where(mask, gates, 0.0)
    g = g / (jnp.sum(g, axis=-1, keepdims=True) + 1e-9)
    qh = jnp.dot(_ln(th, hg_ref[...], hb_ref[...]), qhw_ref[...],
                 preferred_element_type=jnp.float32) + qhb_ref[...]
    sh = jnp.dot(qh, kh_ref[...].T, preferred_element_type=jnp.float32) * (c3 ** -0.5)
    sh = sh.reshape(tn, n_exp, m_dict)
    sh = sh - jnp.max(sh, axis=-1, keepdims=True)
    eh = jnp.exp(sh)
    ph = eh / jnp.sum(eh, axis=-1, keepdims=True)
    wh = (ph * g[..., None]).reshape(tn, n_exp * m_dict)
    tho_ref[...] = th + jnp.dot(wh, dh_ref[...], preferred_element_type=jnp.float32)


def _token_stage(tok_l, tok_h, kl, dl, kh, dh, p, tile=896):
    n, c = tok_l.shape
    c3 = tok_h.shape[1]
    n_exp, m_dict = p['dict_high'].shape[0], p['dict_high'].shape[1]
    em = n_exp * m_dict
    hmid = p['r_w2'].shape[1]
    grid = n // tile
    body = functools.partial(_token_body, n_exp=n_exp, m_dict=m_dict)
    return pl.pallas_call(
        body,
        grid=(grid,),
        in_specs=[
            pl.BlockSpec((tile, c), lambda i: (i, 0)),
            pl.BlockSpec((tile, c3), lambda i: (i, 0)),
            _full((1, c)), _full((1, c)), _full((c, c)), _full((1, c)),
            _full((m_dict, c)), _full((m_dict, c)),
            _full((c3 + c, c3)), _full((1, c3)),
            _full((c3, hmid)), _full((1, hmid)), _full((hmid, n_exp)), _full((1, n_exp)),
            _full((1, c3)), _full((1, c3)), _full((c3, c3)), _full((1, c3)),
            _full((em, c3)), _full((em, c3)),
        ],
        out_specs=[
            pl.BlockSpec((tile, c), lambda i: (i, 0)),
            pl.BlockSpec((tile, c3), lambda i: (i, 0)),
        ],
        out_shape=(
            jax.ShapeDtypeStruct((n, c), jnp.float32),
            jax.ShapeDtypeStruct((n, c3), jnp.float32),
        ),
    )(
        tok_l, tok_h,
        p['ln_low_g'].reshape(1, c), p['ln_low_b'].reshape(1, c),
        p['q_low_w'], p['q_low_b'].reshape(1, c), kl, dl,
        p['r_w1'], p['r_b1'].reshape(1, c3),
        p['r_w2'], p['r_b2'].reshape(1, hmid), p['r_w3'], p['r_b3'].reshape(1, n_exp),
        p['ln_high_g'].reshape(1, c3), p['ln_high_b'].reshape(1, c3),
        p['q_high_w'], p['q_high_b'].reshape(1, c3), kh, dh,
    )


def _vinv_body(ll_ref, th_ref,
               pdww_ref, pdwb_ref, ppwt_ref, ppwb_ref,
               udww_ref, udwb_ref, upwt_ref, upwb_ref,
               lo_ref, ho_ref, *, tile, nrows, c):
    i = pl.program_id(0)
    ng = pl.num_programs(0)
    start = i * tile
    pp = (pdww_ref, pdwb_ref, ppwt_ref, ppwb_ref)
    pu = (udww_ref, udwb_ref, upwt_ref, upwb_ref)

    ll_w = _window1(ll_ref, start, tile, nrows)
    th_w4 = _window(th_ref, start, tile, nrows)
    a_lh4 = th_w4[..., :c]
    a_hl_w = th_w4[1:-1, :, c:2 * c]
    a_hh4 = th_w4[..., 2 * c:]

    ev_l_ext, od_l = _inv_pair(ll_w, a_lh4, pp, pu, i, ng)
    ev_h_ext, od_h = _inv_pair(a_hl_w, a_hh4, pp, pu, i, ng)

    lo_ref[...] = jnp.stack([ev_l_ext[1:-1], od_l], axis=1)
    ho_ref[...] = jnp.stack([ev_h_ext[1:-1], od_h], axis=1)


def _vinv(ll2, th2, p, tile=28):
    nrows, w, c = ll2.shape
    grid = nrows // tile
    pw = _wts(p['idwt']['P_h_v'], c)
    uw = _wts(p['idwt']['U_h_v'], c)
    body = functools.partial(_vinv_body, tile=tile, nrows=nrows, c=c)
    return pl.pallas_call(
        body,
        grid=(grid,),
        in_specs=[
            _full((nrows, w, c)), _full((nrows, w, 3 * c)),
            _full((9, c)), _full((1, c)), _full((c, c)), _full((1, c)),
            _full((9, c)), _full((1, c)), _full((c, c)), _full((1, c)),
        ],
        out_specs=[
            pl.BlockSpec((tile, 2, w, c), lambda i: (i, 0, 0, 0)),
            pl.BlockSpec((tile, 2, w, c), lambda i: (i, 0, 0, 0)),
        ],
        out_shape=(
            jax.ShapeDtypeStruct((nrows, 2, w, c), jnp.float32),
            jax.ShapeDtypeStruct((nrows, 2, w, c), jnp.float32),
        ),
    )(ll2, th2, *pw, *uw)


def _hinv_body(lv_ref, hv_ref,
               pdww_ref, pdwb_ref, ppwt_ref, ppwb_ref,
               udww_ref, udwb_ref, upwt_ref, upwb_ref,
               ow_ref, ob_ref, o_ref, *, tile, nrows, dout):
    i = pl.program_id(0)
    ng = pl.num_programs(0)
    start = i * tile
    t, w, _ = lv_ref.shape[0], lv_ref.shape[1], 0
    pp = (pdww_ref, pdwb_ref, ppwt_ref, ppwb_ref)
    pu = (udww_ref, udwb_ref, upwt_ref, upwb_ref)

    lo_w = _window1(lv_ref, start, tile, nrows)
    hi_w = _window(hv_ref, start, tile, nrows)
    ev_ext, od = _inv_pair(lo_w, hi_w, pp, pu, i, ng)
    ev = ev_ext[1:-1]
    tw = tile * ev.shape[1]
    c = ev.shape[2]
    ye = jnp.dot(ev.reshape(tw, c), ow_ref[...],
                 preferred_element_type=jnp.float32) + ob_ref[...]
    yo = jnp.dot(od.reshape(tw, c), ow_ref[...],
                 preferred_element_type=jnp.float32) + ob_ref[...]
    o_ref[...] = jnp.concatenate(
        [ye.reshape(tile, ev.shape[1], dout), yo.reshape(tile, ev.shape[1], dout)],
        axis=-1)


def _hinv(lv2, hv2, p, tile=28):
    nrows, w, c = lv2.shape
    dout = p['out_w'].shape[1]
    grid = nrows // tile
    pw = _wts(p['idwt']['P_h_h'], c)
    uw = _wts(p['idwt']['U_h_h'], c)
    body = functools.partial(_hinv_body, tile=tile, nrows=nrows, dout=dout)
    return pl.pallas_call(
        body,
        grid=(grid,),
        in_specs=[
            _full((nrows, w, c)), _full((nrows, w, c)),
            _full((9, c)), _full((1, c)), _full((c, c)), _full((1, c)),
            _full((9, c)), _full((1, c)), _full((c, c)), _full((1, c)),
            _full((c, dout)), _full((1, dout)),
        ],
        out_specs=pl.BlockSpec((tile, w, 2 * dout), lambda i: (i, 0, 0)),
        out_shape=jax.ShapeDtypeStruct((nrows, w, 2 * dout), jnp.float32),
    )(lv2, hv2, *pw, *uw, p['out_w'], p['out_b'].reshape(1, dout))


def kernel(x, params):
    p = params
    b, h, w, din = x.shape
    c = p['x_trans_w'].shape[1]
    c3 = 3 * c
    w2 = w // 2
    h2 = h // 2
    n = h2 * w2

    x2 = x.reshape(h, w2, 2 * din)
    h_horz, l_horz = _hdwt(x2, p)
    ll, th = _vdwt(l_horz.reshape(h2, 2, w2, c),
                   h_horz.reshape(h2, 2, w2, c), p)

    dl, kl, dh, kh = _dict_prep(p)
    tl2, th2 = _token_stage(ll.reshape(n, c), th.reshape(n, c3), kl, dl, kh, dh, p)

    lo, ho = _vinv(tl2.reshape(h2, w2, c), th2.reshape(h2, w2, c3), p)
    y = _hinv(lo.reshape(h, w2, c), ho.reshape(h, w2, c), p)
    return y.reshape(b, h, w, p['out_w'].shape[1])
